# baseline (speedup 1.0000x reference)
import jax
import jax.numpy as jnp
from jax.experimental import pallas as pl
from jax.experimental.pallas import tpu as pltpu

N_ROWS = 65536
IN_DIM = 10
OUT_DIM = 150
BLOCK_M = 16384


def _matmul_block(xt_ref, w_ref, o_ref):
    o_ref[...] = jax.lax.dot_general(
        xt_ref[...],
        w_ref[...],
        dimension_numbers=(((0,), (0,)), ((), ())),
        preferred_element_type=jnp.float32,
    )


@jax.jit
def kernel(sparse_matrix, dense_matrix):
    xt = sparse_matrix.T
    return pl.pallas_call(
        _matmul_block,
        grid=(N_ROWS // BLOCK_M,),
        in_specs=[
            pl.BlockSpec((IN_DIM, BLOCK_M), lambda i: (0, i)),
            pl.BlockSpec((IN_DIM, OUT_DIM), lambda i: (0, 0)),
        ],
        out_specs=pl.BlockSpec((BLOCK_M, OUT_DIM), lambda i: (i, 0)),
        out_shape=jax.ShapeDtypeStruct((N_ROWS, OUT_DIM), jnp.float32),
        compiler_params=pltpu.CompilerParams(
            dimension_semantics=("parallel",),
        ),
    )(xt, dense_matrix)


# speedup vs baseline: 1.0258x; 1.0258x over previous
import jax
import jax.numpy as jnp
from jax.experimental import pallas as pl
from jax.experimental.pallas import tpu as pltpu

N_ROWS = 65536
IN_DIM = 10
OUT_DIM = 150
BLOCK_M = 8192


def _matmul_block(xt_ref, w_ref, o_ref):
    o_ref[...] = jax.lax.dot_general(
        xt_ref[...],
        w_ref[...],
        dimension_numbers=(((0,), (0,)), ((), ())),
        preferred_element_type=jnp.float32,
    )


@jax.jit
def kernel(sparse_matrix, dense_matrix):
    xt = sparse_matrix.T
    return pl.pallas_call(
        _matmul_block,
        grid=(N_ROWS // BLOCK_M,),
        in_specs=[
            pl.BlockSpec((IN_DIM, BLOCK_M), lambda i: (0, i)),
            pl.BlockSpec((IN_DIM, OUT_DIM), lambda i: (0, 0)),
        ],
        out_specs=pl.BlockSpec((BLOCK_M, OUT_DIM), lambda i: (i, 0)),
        out_shape=jax.ShapeDtypeStruct((N_ROWS, OUT_DIM), jnp.float32),
        compiler_params=pltpu.CompilerParams(
            dimension_semantics=("parallel",),
        ),
    )(xt, dense_matrix)
